# baseline (speedup 1.0000x reference)
import functools

import jax
import jax.numpy as jnp
from jax import lax
from jax.experimental import pallas as pl
from jax.experimental.pallas import tpu as pltpu
from jax.experimental.pallas import tpu_sc as plsc

_EMB = 32
_PACK = 128 // _EMB


_BKT = 8192
_QB = _BKT // _PACK


def _transpose_pack(table):
    V, D = table.shape
    grid = -(-V // _BKT)

    def body(tin_ref, out_ref):
        limit = V - pl.program_id(0) * _BKT
        colid = lax.broadcasted_iota(jnp.int32, (D, _BKT), 1)
        tin = jnp.where(colid < limit, tin_ref[...], 0.0)
        rows = lax.broadcasted_iota(jnp.int32, (D, _PACK * D), 0)
        cols = lax.broadcasted_iota(jnp.int32, (D, _PACK * D), 1)
        acc = None
        for q in range(_PACK):
            sel = (cols == rows + q * D).astype(jnp.float32)
            part = lax.dot_general(tin[:, q * _QB:(q + 1) * _QB], sel,
                                   (((0,), (0,)), ((), ())),
                                   preferred_element_type=jnp.float32)
            acc = part if acc is None else acc + part
        out_ref[...] = acc

    return pl.pallas_call(
        body,
        grid=(grid,),
        in_specs=[pl.BlockSpec((D, _BKT), lambda i: (0, i))],
        out_specs=pl.BlockSpec((_QB, _PACK * D), lambda i: (i, 0)),
        out_shape=jax.ShapeDtypeStruct((grid * _QB, _PACK * D), jnp.float32),
    )(table.T)


def _sc_gather(packed, pidx):
    B = pidx.shape[0]
    W = packed.shape[1]
    info = plsc.get_sparse_core_info()
    nw = info.num_cores * info.num_subcores
    b_per_w = B // nw
    mesh = plsc.VectorSubcoreMesh(core_axis_name="c", subcore_axis_name="s")

    @functools.partial(
        pl.kernel,
        mesh=mesh,
        out_type=jax.ShapeDtypeStruct((B, W), jnp.float32),
        scratch_types=[
            pltpu.VMEM((b_per_w + 16,), jnp.int32),
            pltpu.VMEM((b_per_w, W), jnp.float32),
            pltpu.SemaphoreType.DMA,
        ],
    )
    def gather_kernel(packed_hbm, pidx_hbm, out_hbm, pidx_v, pk_v, sem):
        wid = lax.axis_index("s") * info.num_cores + lax.axis_index("c")
        base = wid * b_per_w
        pltpu.sync_copy(pidx_hbm.at[pl.ds(base, b_per_w)],
                        pidx_v.at[pl.ds(0, b_per_w)])

        @pl.loop(0, b_per_w, step=16)
        def _(i):
            kv = pidx_v[pl.ds(i, 16)]
            for j in range(16):
                pltpu.async_copy(packed_hbm.at[pl.ds(kv[j], 1)],
                                 pk_v.at[pl.ds(i + j, 1)], sem)

        pltpu.make_async_copy(packed_hbm.at[pl.ds(0, b_per_w)], pk_v,
                              sem).wait()
        pltpu.sync_copy(pk_v, out_hbm.at[pl.ds(base, b_per_w)])

    return gather_kernel(packed, pidx)


def _mlp_body(schp_ref, qoh_ref, g_ref, go_ref, su_ref, me_ref,
              gt_ref, got_ref, sut_ref, met_ref,
              w1_ref, b1_ref, w2_ref, b2_ref, w3_ref, b3_ref, out_ref):
    bk = g_ref.shape[0]
    bf = jnp.bfloat16

    def emb(idx_ref, tab_ref):
        v = tab_ref.shape[0]
        oh_t = (idx_ref[...][None, :] == lax.broadcasted_iota(jnp.int32, (v, bk), 0))
        e = lax.dot_general(oh_t.astype(bf), tab_ref[...].astype(bf),
                            (((0,), (0,)), ((), ())),
                            preferred_element_type=jnp.float32)
        return e.astype(bf)

    schp = schp_ref[...]
    qoh = qoh_ref[...]
    sch = jnp.where(qoh[:, 0:1] > 0.5, schp[:, 0:_EMB], 0.0)
    for q in range(1, _PACK):
        sch = sch + jnp.where(qoh[:, q:q + 1] > 0.5,
                              schp[:, q * _EMB:(q + 1) * _EMB], 0.0)

    x = jnp.concatenate(
        [sch.astype(bf), emb(g_ref, gt_ref), emb(go_ref, got_ref),
         emb(su_ref, sut_ref), emb(me_ref, met_ref)], axis=1)
    h = jnp.maximum(
        jnp.dot(x, w1_ref[...].astype(bf), preferred_element_type=jnp.float32)
        + b1_ref[...], 0.0)
    h = jnp.maximum(
        jnp.dot(h.astype(bf), w2_ref[...].astype(bf),
                preferred_element_type=jnp.float32)
        + b2_ref[...], 0.0)
    out_ref[...] = (
        jnp.dot(h.astype(bf), w3_ref[...].astype(bf),
                preferred_element_type=jnp.float32) + b3_ref[...]).T


def _pad_rows(t):
    v = t.shape[0]
    vp = -(-v // 8) * 8
    return jnp.pad(t, ((0, vp - v), (0, 0)))


def _tc_mlp_t(schp, qoh, g, go, su, me, gt, got, sut, met,
              W1, b1, W2, b2, W3, b3):
    B, D = schp.shape
    BK = 2048
    grid = B // BK
    H1, H2, DO = W1.shape[1], W2.shape[1], W3.shape[1]

    def blk(i, *_):
        return (i, 0)

    def blkT(i, *_):
        return (0, i)

    def blk1(i, *_):
        return (i,)

    def rep(*_):
        return (0, 0)

    return pl.pallas_call(
        _mlp_body,
        grid=(grid,),
        in_specs=[
            pl.BlockSpec((BK, D), blk),
            pl.BlockSpec((BK, _PACK), blk),
            pl.BlockSpec((BK,), blk1),
            pl.BlockSpec((BK,), blk1),
            pl.BlockSpec((BK,), blk1),
            pl.BlockSpec((BK,), blk1),
            pl.BlockSpec(gt.shape, rep),
            pl.BlockSpec(got.shape, rep),
            pl.BlockSpec(sut.shape, rep),
            pl.BlockSpec(met.shape, rep),
            pl.BlockSpec(W1.shape, rep),
            pl.BlockSpec((1, H1), rep),
            pl.BlockSpec(W2.shape, rep),
            pl.BlockSpec((1, H2), rep),
            pl.BlockSpec(W3.shape, rep),
            pl.BlockSpec((1, DO), rep),
        ],
        out_specs=pl.BlockSpec((DO, BK), blkT),
        out_shape=jax.ShapeDtypeStruct((DO, B), jnp.float32),
    )(schp, qoh, g, go, su, me, gt, got, sut, met,
      W1, b1.reshape(1, H1), W2, b2.reshape(1, H2), W3, b3.reshape(1, DO))


def kernel(school_idx, grade_idx, goal_idx, subject_idx, method_idx,
           school_table, grade_table, goal_table, subject_table, method_table,
           W1, b1, W2, b2, W3, b3):
    k32 = school_idx.astype(jnp.int32)
    pidx = ((k32 >> 13) << 11) + (k32 & (_QB - 1))
    qoh = (
        ((k32 >> 11) & (_PACK - 1))[:, None]
        == jnp.arange(_PACK, dtype=jnp.int32)[None, :]
    ).astype(jnp.float32)
    schp = _sc_gather(_transpose_pack(school_table), pidx)
    out_t = _tc_mlp_t(
        schp, qoh,
        grade_idx.astype(jnp.int32), goal_idx.astype(jnp.int32),
        subject_idx.astype(jnp.int32), method_idx.astype(jnp.int32),
        _pad_rows(grade_table), _pad_rows(goal_table),
        _pad_rows(subject_table), _pad_rows(method_table),
        W1, b1, W2, b2, W3, b3)
    return out_t.T


# speedup vs baseline: 1.2762x; 1.2762x over previous
import functools

import jax
import jax.numpy as jnp
from jax import lax
from jax.experimental import pallas as pl
from jax.experimental.pallas import tpu as pltpu
from jax.experimental.pallas import tpu_sc as plsc

_EMB = 32
_PACK = 128 // _EMB


def _transpose_table(table):
    V, D = table.shape
    BKT = 8192
    grid = -(-V // BKT)

    def body(tin_ref, out_ref):
        out_ref[...] = tin_ref[...].T

    return pl.pallas_call(
        body,
        grid=(grid,),
        in_specs=[pl.BlockSpec((D, BKT), lambda i: (0, i))],
        out_specs=pl.BlockSpec((BKT, D), lambda i: (i, 0)),
        out_shape=jax.ShapeDtypeStruct((V, D), jnp.float32),
    )(table.T)


def _sc_gather(packed, pidx):
    B = pidx.shape[0]
    W = packed.shape[1]
    info = plsc.get_sparse_core_info()
    nw = info.num_cores * info.num_subcores
    b_per_w = B // nw
    mesh = plsc.VectorSubcoreMesh(core_axis_name="c", subcore_axis_name="s")

    @functools.partial(
        pl.kernel,
        mesh=mesh,
        out_type=jax.ShapeDtypeStruct((B, W), jnp.float32),
        scratch_types=[
            pltpu.VMEM((b_per_w + 16,), jnp.int32),
            pltpu.VMEM((b_per_w, W), jnp.float32),
            pltpu.SemaphoreType.DMA,
        ],
    )
    def gather_kernel(packed_hbm, pidx_hbm, out_hbm, pidx_v, pk_v, sem):
        wid = lax.axis_index("s") * info.num_cores + lax.axis_index("c")
        base = wid * b_per_w
        pltpu.sync_copy(pidx_hbm.at[pl.ds(base, b_per_w)],
                        pidx_v.at[pl.ds(0, b_per_w)])

        @pl.loop(0, b_per_w, step=16)
        def _(i):
            kv = pidx_v[pl.ds(i, 16)]
            for j in range(16):
                pltpu.async_copy(packed_hbm.at[pl.ds(kv[j], 1)],
                                 pk_v.at[pl.ds(i + j, 1)], sem)

        pltpu.make_async_copy(packed_hbm.at[pl.ds(0, b_per_w)], pk_v,
                              sem).wait()
        pltpu.sync_copy(pk_v, out_hbm.at[pl.ds(base, b_per_w)])

    return gather_kernel(packed, pidx)


def _mlp_body(sch_ref, g_ref, go_ref, su_ref, me_ref,
              gt_ref, got_ref, sut_ref, met_ref,
              w1_ref, b1_ref, w2_ref, b2_ref, w3_ref, b3_ref, out_ref):
    bk = g_ref.shape[0]
    bf = jnp.bfloat16

    def emb(idx_ref, tab_ref):
        v = tab_ref.shape[0]
        oh_t = (idx_ref[...][None, :] == lax.broadcasted_iota(jnp.int32, (v, bk), 0))
        e = lax.dot_general(oh_t.astype(bf), tab_ref[...].astype(bf),
                            (((0,), (0,)), ((), ())),
                            preferred_element_type=jnp.float32)
        return e.astype(bf)

    x = jnp.concatenate(
        [sch_ref[...].astype(bf), emb(g_ref, gt_ref), emb(go_ref, got_ref),
         emb(su_ref, sut_ref), emb(me_ref, met_ref)], axis=1)
    h = jnp.maximum(
        jnp.dot(x, w1_ref[...].astype(bf), preferred_element_type=jnp.float32)
        + b1_ref[...], 0.0)
    h = jnp.maximum(
        jnp.dot(h.astype(bf), w2_ref[...].astype(bf),
                preferred_element_type=jnp.float32)
        + b2_ref[...], 0.0)
    out_ref[...] = (
        jnp.dot(h.astype(bf), w3_ref[...].astype(bf),
                preferred_element_type=jnp.float32) + b3_ref[...]).T


def _pad_rows(t):
    v = t.shape[0]
    vp = -(-v // 8) * 8
    return jnp.pad(t, ((0, vp - v), (0, 0)))


def _tc_mlp_t(sch, g, go, su, me, gt, got, sut, met,
              W1, b1, W2, b2, W3, b3):
    B, D = sch.shape
    BK = 2048
    grid = B // BK
    H1, H2, DO = W1.shape[1], W2.shape[1], W3.shape[1]

    def blk(i, *_):
        return (i, 0)

    def blkT(i, *_):
        return (0, i)

    def blk1(i, *_):
        return (i,)

    def rep(*_):
        return (0, 0)

    return pl.pallas_call(
        _mlp_body,
        grid=(grid,),
        in_specs=[
            pl.BlockSpec((BK, D), blk),
            pl.BlockSpec((BK,), blk1),
            pl.BlockSpec((BK,), blk1),
            pl.BlockSpec((BK,), blk1),
            pl.BlockSpec((BK,), blk1),
            pl.BlockSpec(gt.shape, rep),
            pl.BlockSpec(got.shape, rep),
            pl.BlockSpec(sut.shape, rep),
            pl.BlockSpec(met.shape, rep),
            pl.BlockSpec(W1.shape, rep),
            pl.BlockSpec((1, H1), rep),
            pl.BlockSpec(W2.shape, rep),
            pl.BlockSpec((1, H2), rep),
            pl.BlockSpec(W3.shape, rep),
            pl.BlockSpec((1, DO), rep),
        ],
        out_specs=pl.BlockSpec((DO, BK), blkT),
        out_shape=jax.ShapeDtypeStruct((DO, B), jnp.float32),
    )(sch, g, go, su, me, gt, got, sut, met,
      W1, b1.reshape(1, H1), W2, b2.reshape(1, H2), W3, b3.reshape(1, DO))


def kernel(school_idx, grade_idx, goal_idx, subject_idx, method_idx,
           school_table, grade_table, goal_table, subject_table, method_table,
           W1, b1, W2, b2, W3, b3):
    sch = _sc_gather(_transpose_table(school_table),
                     school_idx.astype(jnp.int32))
    out_t = _tc_mlp_t(
        sch,
        grade_idx.astype(jnp.int32), goal_idx.astype(jnp.int32),
        subject_idx.astype(jnp.int32), method_idx.astype(jnp.int32),
        _pad_rows(grade_table), _pad_rows(goal_table),
        _pad_rows(subject_table), _pad_rows(method_table),
        W1, b1, W2, b2, W3, b3)
    return out_t.T


# speedup vs baseline: 1.3016x; 1.0199x over previous
import functools

import jax
import jax.numpy as jnp
from jax import lax
from jax.experimental import pallas as pl
from jax.experimental.pallas import tpu as pltpu
from jax.experimental.pallas import tpu_sc as plsc

_EMB = 32
_PACK = 128 // _EMB


def _transpose_table(table):
    V, D = table.shape
    BKT = 8192
    grid = -(-V // BKT)

    def body(tin_ref, out_ref):
        out_ref[...] = tin_ref[...].T

    return pl.pallas_call(
        body,
        grid=(grid,),
        in_specs=[pl.BlockSpec((D, BKT), lambda i: (0, i))],
        out_specs=pl.BlockSpec((BKT, D), lambda i: (i, 0)),
        out_shape=jax.ShapeDtypeStruct((V, D), jnp.float32),
    )(table.T)


def _sc_gather(packed, pidx):
    B = pidx.shape[0]
    W = packed.shape[1]
    info = plsc.get_sparse_core_info()
    nw = info.num_cores * info.num_subcores
    b_per_w = B // nw
    mesh = plsc.VectorSubcoreMesh(core_axis_name="c", subcore_axis_name="s")

    @functools.partial(
        pl.kernel,
        mesh=mesh,
        out_type=jax.ShapeDtypeStruct((B, W), jnp.float32),
        scratch_types=[
            pltpu.VMEM((b_per_w + 16,), jnp.int32),
            pltpu.VMEM((b_per_w, W), jnp.float32),
            pltpu.SemaphoreType.DMA,
        ],
    )
    def gather_kernel(packed_hbm, pidx_hbm, out_hbm, pidx_v, pk_v, sem):
        wid = lax.axis_index("s") * info.num_cores + lax.axis_index("c")
        base = wid * b_per_w
        pltpu.sync_copy(pidx_hbm.at[pl.ds(base, b_per_w)],
                        pidx_v.at[pl.ds(0, b_per_w)])

        @pl.loop(0, b_per_w, step=16)
        def _(i):
            kv = pidx_v[pl.ds(i, 16)]
            for j in range(16):
                pltpu.async_copy(packed_hbm.at[pl.ds(kv[j], 1)],
                                 pk_v.at[pl.ds(i + j, 1)], sem)

        pltpu.make_async_copy(packed_hbm.at[pl.ds(0, b_per_w)], pk_v,
                              sem).wait()
        pltpu.sync_copy(pk_v, out_hbm.at[pl.ds(base, b_per_w)])

    return gather_kernel(packed, pidx)


def _mlp_body(sch_ref, g_ref, go_ref, su_ref, me_ref,
              gt_ref, got_ref, sut_ref, met_ref,
              w1_ref, b1_ref, w2_ref, b2_ref, w3_ref, b3_ref, out_ref):
    bk = g_ref.shape[0]
    bf = jnp.bfloat16

    def emb(idx_ref, tab_ref):
        v = tab_ref.shape[0]
        oh_t = (idx_ref[...][None, :] == lax.broadcasted_iota(jnp.int32, (v, bk), 0))
        e = lax.dot_general(oh_t.astype(bf), tab_ref[...].astype(bf),
                            (((0,), (0,)), ((), ())),
                            preferred_element_type=jnp.float32)
        return e.astype(bf)

    x = jnp.concatenate(
        [sch_ref[...].astype(bf), emb(g_ref, gt_ref), emb(go_ref, got_ref),
         emb(su_ref, sut_ref), emb(me_ref, met_ref)], axis=1)
    h = jnp.maximum(
        jnp.dot(x, w1_ref[...].astype(bf), preferred_element_type=jnp.float32)
        + b1_ref[...], 0.0)
    h = jnp.maximum(
        jnp.dot(h.astype(bf), w2_ref[...].astype(bf),
                preferred_element_type=jnp.float32)
        + b2_ref[...], 0.0)
    out_ref[...] = (
        jnp.dot(h.astype(bf), w3_ref[...].astype(bf),
                preferred_element_type=jnp.float32) + b3_ref[...]).T


def _pad_rows(t):
    v = t.shape[0]
    vp = -(-v // 8) * 8
    return jnp.pad(t, ((0, vp - v), (0, 0)))


def _tc_mlp_t(sch, g, go, su, me, gt, got, sut, met,
              W1, b1, W2, b2, W3, b3):
    B, D = sch.shape
    BK = 4096
    grid = B // BK
    H1, H2, DO = W1.shape[1], W2.shape[1], W3.shape[1]

    def blk(i, *_):
        return (i, 0)

    def blkT(i, *_):
        return (0, i)

    def blk1(i, *_):
        return (i,)

    def rep(*_):
        return (0, 0)

    return pl.pallas_call(
        _mlp_body,
        grid=(grid,),
        in_specs=[
            pl.BlockSpec((BK, D), blk),
            pl.BlockSpec((BK,), blk1),
            pl.BlockSpec((BK,), blk1),
            pl.BlockSpec((BK,), blk1),
            pl.BlockSpec((BK,), blk1),
            pl.BlockSpec(gt.shape, rep),
            pl.BlockSpec(got.shape, rep),
            pl.BlockSpec(sut.shape, rep),
            pl.BlockSpec(met.shape, rep),
            pl.BlockSpec(W1.shape, rep),
            pl.BlockSpec((1, H1), rep),
            pl.BlockSpec(W2.shape, rep),
            pl.BlockSpec((1, H2), rep),
            pl.BlockSpec(W3.shape, rep),
            pl.BlockSpec((1, DO), rep),
        ],
        out_specs=pl.BlockSpec((DO, BK), blkT),
        out_shape=jax.ShapeDtypeStruct((DO, B), jnp.float32),
    )(sch, g, go, su, me, gt, got, sut, met,
      W1, b1.reshape(1, H1), W2, b2.reshape(1, H2), W3, b3.reshape(1, DO))


def kernel(school_idx, grade_idx, goal_idx, subject_idx, method_idx,
           school_table, grade_table, goal_table, subject_table, method_table,
           W1, b1, W2, b2, W3, b3):
    sch = _sc_gather(_transpose_table(school_table),
                     school_idx.astype(jnp.int32))
    out_t = _tc_mlp_t(
        sch,
        grade_idx.astype(jnp.int32), goal_idx.astype(jnp.int32),
        subject_idx.astype(jnp.int32), method_idx.astype(jnp.int32),
        _pad_rows(grade_table), _pad_rows(goal_table),
        _pad_rows(subject_table), _pad_rows(method_table),
        W1, b1, W2, b2, W3, b3)
    return out_t.T
